# baseline (speedup 1.0000x reference)
import functools

import jax
import jax.numpy as jnp
from jax import lax
from jax.experimental import pallas as pl
from jax.experimental.pallas import tpu as pltpu
from jax.experimental.pallas import tpu_sc as plsc

NUM_PL = 2000
NUM_TR = 8000
D = 128
N = NUM_PL + NUM_TR
N_PAD = 10240
E = 320000
NS = 16
CHUNK = 128
NC = 2
NCH = 80
NCHT = NC * NS * NCH
E_PAD = NCHT * CHUNK
ROWS_PER_TILE = N_PAD // NS

_mesh = plsc.VectorSubcoreMesh(core_axis_name="c", subcore_axis_name="s")


@functools.partial(
    pl.kernel,
    mesh=_mesh,
    compiler_params=pltpu.CompilerParams(
        needs_layout_passes=False, use_tc_tiling_on_sc=False),
    out_type=jax.ShapeDtypeStruct((NC, N_PAD, D), jnp.float32),
    scratch_types=[
        pltpu.VMEM((4, CHUNK), jnp.int32),
        pltpu.VMEM((4, CHUNK), jnp.int32),
        pltpu.VMEM((4, CHUNK), jnp.float32),
        pltpu.VMEM((CHUNK, D // 2), jnp.int32),
        pltpu.VMEM((CHUNK, D // 2), jnp.int32),
        pltpu.VMEM((CHUNK, D), jnp.float32),
        pltpu.VMEM_SHARED((N_PAD, D), jnp.float32),
        pltpu.SemaphoreType.DMA,
        pltpu.SemaphoreType.DMA,
        pltpu.SemaphoreType.DMA,
        pltpu.SemaphoreType.DMA,
        pltpu.SemaphoreType.DMA,
        pltpu.SemaphoreType.DMA,
    ],
)
def _propagate(emb_hbm, src_hbm, dst_hbm, w_hbm, out_hbm,
               srci, dsti, wvi, rows0, rows1, rowsf, acc,
               g0, g1, st0, st1, st2, st3):
    c = lax.axis_index("c")
    s = lax.axis_index("s")
    wbase = (c * NS + s) * NCH

    zv = jnp.zeros((16,), jnp.float32)

    def zero_body(k, carry):
        for g in range(D // 16):
            rowsf[k, pl.ds(g * 16, 16)] = zv
        return carry

    lax.fori_loop(0, CHUNK, zero_body, 0, unroll=4)
    for r in range(ROWS_PER_TILE // CHUNK):
        pltpu.sync_copy(
            rowsf, acc.at[pl.ds(s * ROWS_PER_TILE + r * CHUNK, CHUNK)])
    plsc.subcore_barrier()

    rows_ring = (rows0, rows1)
    gsem_ring = (g0, g1)
    stsem_ring = (st0, st1, st2, st3)

    def stage_start(j, slot):
        sem = stsem_ring[slot]
        pltpu.async_copy(src_hbm.at[wbase + j], srci.at[slot], sem)
        pltpu.async_copy(dst_hbm.at[wbase + j], dsti.at[slot], sem)
        pltpu.async_copy(w_hbm.at[wbase + j], wvi.at[slot], sem)

    def stage_wait(j, slot):
        sem = stsem_ring[slot]
        pltpu.make_async_copy(src_hbm.at[wbase + j], srci.at[slot], sem).wait()
        pltpu.make_async_copy(dst_hbm.at[wbase + j], dsti.at[slot], sem).wait()
        pltpu.make_async_copy(w_hbm.at[wbase + j], wvi.at[slot], sem).wait()

    iota2 = lax.iota(jnp.int32, 16) * 2
    cols_lo = [iota2 + (g * 32) for g in range(D // 32)]
    cols_hi = [iota2 + (g * 32 + 1) for g in range(D // 32)]
    zeros16 = jnp.zeros((16,), jnp.int32)

    def scale_chunk(rows, slot):
        jv = jnp.full((16,), slot, jnp.int32)

        def edge_body(e, carry):
            wb = plsc.load_gather(wvi, [jv, zeros16 + e])
            ev = zeros16 + e
            for g in range(D // 32):
                xi = rows[e, pl.ds(g * 16, 16)]
                lo = plsc.bitcast(xi << 16, jnp.float32) * wb
                hi = plsc.bitcast(xi & jnp.int32(-65536), jnp.float32) * wb
                plsc.store_scatter(rowsf, [ev, cols_lo[g]], lo)
                plsc.store_scatter(rowsf, [ev, cols_hi[g]], hi)
            return carry

        lax.fori_loop(0, CHUNK, edge_body, 0, unroll=4)

    stage_start(0, 0)
    stage_start(1, 1)
    stage_wait(0, 0)
    pltpu.async_copy(emb_hbm.at[srci.at[0]], rows0, g0)

    def quad_body(i, carry):
        for u in range(4):
            j = 4 * i + u
            rb, gb = rows_ring[u % 2], gsem_ring[u % 2]
            pltpu.make_async_copy(emb_hbm.at[srci.at[u]], rb, gb).wait()

            @pl.when(j + 2 < NCH)
            def _():
                stage_start(j + 2, (u + 2) % 4)

            @pl.when(j + 1 < NCH)
            def _():
                stage_wait(j + 1, (u + 1) % 4)
                pltpu.async_copy(emb_hbm.at[srci.at[(u + 1) % 4]],
                                 rows_ring[(u + 1) % 2], gsem_ring[(u + 1) % 2])

            scale_chunk(rb, u)
            pltpu.sync_copy(rowsf, acc.at[dsti.at[u]], add=True)
        return carry

    lax.fori_loop(0, NCH // 4, quad_body, 0)
    plsc.subcore_barrier()
    sl = pl.ds(s * ROWS_PER_TILE, ROWS_PER_TILE)
    pltpu.sync_copy(acc.at[sl], out_hbm.at[c, sl])


_BLK = N_PAD // 8


def _combine_body(p_ref, t_ref, emb_ref, tot_ref):
    ssum = p_ref[0] + p_ref[1]
    emb_ref[...] = ssum.astype(jnp.bfloat16)
    tot_ref[...] = t_ref[...] + ssum


def _combine(parts, total):
    return pl.pallas_call(
        _combine_body,
        grid=(N_PAD // _BLK,),
        in_specs=[
            pl.BlockSpec((NC, _BLK, D), lambda i: (0, i, 0)),
            pl.BlockSpec((_BLK, D), lambda i: (i, 0)),
        ],
        out_specs=[
            pl.BlockSpec((_BLK, D), lambda i: (i, 0)),
            pl.BlockSpec((_BLK, D), lambda i: (i, 0)),
        ],
        out_shape=[
            jax.ShapeDtypeStruct((N_PAD, D), jnp.bfloat16),
            jax.ShapeDtypeStruct((N_PAD, D), jnp.float32),
        ],
    )(parts, total)


def _final_body(p_ref, t_ref, out_ref):
    out_ref[...] = (t_ref[...] + p_ref[0] + p_ref[1]) * 0.25


def _final(parts, total):
    return pl.pallas_call(
        _final_body,
        grid=(N_PAD // _BLK,),
        in_specs=[
            pl.BlockSpec((NC, _BLK, D), lambda i: (0, i, 0)),
            pl.BlockSpec((_BLK, D), lambda i: (i, 0)),
        ],
        out_specs=pl.BlockSpec((_BLK, D), lambda i: (i, 0)),
        out_shape=jax.ShapeDtypeStruct((N_PAD, D), jnp.float32),
    )(parts, total)


def kernel(playlist_weight, track_weight, edge_index, edge_weight):
    emb0 = jnp.zeros((N_PAD, D), jnp.float32)
    emb0 = emb0.at[:NUM_PL].set(playlist_weight)
    emb0 = emb0.at[NUM_PL:N].set(track_weight)
    src = edge_index[0].astype(jnp.int32)
    dst = edge_index[1].astype(jnp.int32)
    w = edge_weight.astype(jnp.float32)
    pad = E_PAD - E
    spread = (jnp.arange(pad, dtype=jnp.int32) * 8) % N_PAD
    src_p = jnp.concatenate([src, spread]).reshape(NCHT, CHUNK)
    dst_p = jnp.concatenate([dst, spread]).reshape(NCHT, CHUNK)
    w_p = jnp.concatenate(
        [w, jnp.zeros((pad,), jnp.float32)]).reshape(NCHT, CHUNK)

    def pack_bf16(x):
        return jax.lax.bitcast_convert_type(
            x.reshape(N_PAD, D // 2, 2), jnp.int32)

    emb = pack_bf16(emb0.astype(jnp.bfloat16))
    total = emb0
    mean = None
    for layer in range(3):
        parts = _propagate(emb, src_p, dst_p, w_p)
        if layer < 2:
            emb_bf, total = _combine(parts, total)
            emb = pack_bf16(emb_bf)
        else:
            mean = _final(parts, total)
    return mean[:NUM_PL], mean[NUM_PL:N]


# speedup vs baseline: 1.9982x; 1.9982x over previous
import functools

import jax
import jax.numpy as jnp
from jax import lax
from jax.experimental import pallas as pl
from jax.experimental.pallas import tpu as pltpu
from jax.experimental.pallas import tpu_sc as plsc

NUM_PL = 2000
NUM_TR = 8000
D = 128
N = NUM_PL + NUM_TR
N_PAD = 10240
E = 320000
NS = 16
CHUNK = 128
NC = 2
NCH = 80
NCHT = NC * NS * NCH
E_PAD = NCHT * CHUNK
ROWS_PER_TILE = N_PAD // NS

_mesh = plsc.VectorSubcoreMesh(core_axis_name="c", subcore_axis_name="s")


@functools.partial(
    pl.kernel,
    mesh=_mesh,
    compiler_params=pltpu.CompilerParams(needs_layout_passes=False),
    out_type=jax.ShapeDtypeStruct((NC, N_PAD, D), jnp.float32),
    scratch_types=[
        pltpu.VMEM((4, CHUNK), jnp.int32),
        pltpu.VMEM((4, CHUNK), jnp.int32),
        pltpu.VMEM((4, CHUNK), jnp.float32),
        pltpu.VMEM((CHUNK, D), jnp.float32),
        pltpu.VMEM((CHUNK, D), jnp.float32),
        pltpu.VMEM_SHARED((N_PAD, D), jnp.float32),
        pltpu.SemaphoreType.DMA,
        pltpu.SemaphoreType.DMA,
        pltpu.SemaphoreType.DMA,
        pltpu.SemaphoreType.DMA,
        pltpu.SemaphoreType.DMA,
        pltpu.SemaphoreType.DMA,
        pltpu.SemaphoreType.DMA,
        pltpu.SemaphoreType.DMA,
    ],
)
def _propagate(emb_hbm, src_hbm, dst_hbm, w_hbm, out_hbm,
               srci, dsti, wvi, rows0, rows1, acc,
               g0, g1, s0, s1, st0, st1, st2, st3):
    c = lax.axis_index("c")
    s = lax.axis_index("s")
    wbase = (c * NS + s) * NCH

    zv = jnp.zeros((16,), jnp.float32)

    def zero_body(k, carry):
        for g in range(D // 16):
            rows0[k, pl.ds(g * 16, 16)] = zv
        return carry

    lax.fori_loop(0, CHUNK, zero_body, 0, unroll=4)
    for r in range(ROWS_PER_TILE // CHUNK):
        pltpu.sync_copy(
            rows0, acc.at[pl.ds(s * ROWS_PER_TILE + r * CHUNK, CHUNK)])
    plsc.subcore_barrier()

    rows_ring = (rows0, rows1)
    gsem_ring = (g0, g1)
    ssem_ring = (s0, s1)
    stsem_ring = (st0, st1, st2, st3)

    def stage_start(j, slot):
        sem = stsem_ring[slot]
        pltpu.async_copy(src_hbm.at[wbase + j], srci.at[slot], sem)
        pltpu.async_copy(dst_hbm.at[wbase + j], dsti.at[slot], sem)
        pltpu.async_copy(w_hbm.at[wbase + j], wvi.at[slot], sem)

    def stage_wait(j, slot):
        sem = stsem_ring[slot]
        pltpu.make_async_copy(src_hbm.at[wbase + j], srci.at[slot], sem).wait()
        pltpu.make_async_copy(dst_hbm.at[wbase + j], dsti.at[slot], sem).wait()
        pltpu.make_async_copy(w_hbm.at[wbase + j], wvi.at[slot], sem).wait()

    zeros16 = jnp.zeros((16,), jnp.int32)

    def scale_chunk(rows, slot):
        jv = jnp.full((16,), slot, jnp.int32)

        def edge_body(e, carry):
            wb = plsc.load_gather(wvi, [jv, zeros16 + e])
            for g in range(D // 16):
                sl = pl.ds(g * 16, 16)
                rows[e, sl] = rows[e, sl] * wb
            return carry

        lax.fori_loop(0, CHUNK, edge_body, 0, unroll=4)

    stage_start(0, 0)
    stage_start(1, 1)
    stage_wait(0, 0)
    pltpu.async_copy(emb_hbm.at[srci.at[0]], rows0, g0)

    def quad_body(i, carry):
        for u in range(4):
            j = 4 * i + u
            b = u % 2
            bn = (u + 1) % 2
            rb = rows_ring[b]
            pltpu.make_async_copy(emb_hbm.at[srci.at[u]], rb,
                                  gsem_ring[b]).wait()

            @pl.when(j + 2 < NCH)
            def _():
                stage_start(j + 2, (u + 2) % 4)

            @pl.when(jnp.logical_and(j >= 1, j + 1 < NCH))
            def _():
                pltpu.make_async_copy(
                    rows_ring[bn], acc.at[dsti.at[(u + 3) % 4]],
                    ssem_ring[bn]).wait()

            @pl.when(j + 1 < NCH)
            def _():
                stage_wait(j + 1, (u + 1) % 4)
                pltpu.async_copy(emb_hbm.at[srci.at[(u + 1) % 4]],
                                 rows_ring[bn], gsem_ring[bn])

            scale_chunk(rb, u)
            pltpu.async_copy(rb, acc.at[dsti.at[u]], ssem_ring[b], add=True)
        return carry

    lax.fori_loop(0, NCH // 4, quad_body, 0)
    pltpu.make_async_copy(rows_ring[0], acc.at[dsti.at[2]], s0).wait()
    pltpu.make_async_copy(rows_ring[1], acc.at[dsti.at[3]], s1).wait()
    plsc.subcore_barrier()
    sl = pl.ds(s * ROWS_PER_TILE, ROWS_PER_TILE)
    pltpu.sync_copy(acc.at[sl], out_hbm.at[c, sl])


_BLK = N_PAD // 8


def _combine_body(p_ref, t_ref, emb_ref, tot_ref):
    ssum = p_ref[0] + p_ref[1]
    emb_ref[...] = ssum
    tot_ref[...] = t_ref[...] + ssum


def _combine(parts, total):
    return pl.pallas_call(
        _combine_body,
        grid=(N_PAD // _BLK,),
        in_specs=[
            pl.BlockSpec((NC, _BLK, D), lambda i: (0, i, 0)),
            pl.BlockSpec((_BLK, D), lambda i: (i, 0)),
        ],
        out_specs=[
            pl.BlockSpec((_BLK, D), lambda i: (i, 0)),
            pl.BlockSpec((_BLK, D), lambda i: (i, 0)),
        ],
        out_shape=[
            jax.ShapeDtypeStruct((N_PAD, D), jnp.float32),
            jax.ShapeDtypeStruct((N_PAD, D), jnp.float32),
        ],
    )(parts, total)


def _final_body(p_ref, t_ref, out_ref):
    out_ref[...] = (t_ref[...] + p_ref[0] + p_ref[1]) * 0.25


def _final(parts, total):
    return pl.pallas_call(
        _final_body,
        grid=(N_PAD // _BLK,),
        in_specs=[
            pl.BlockSpec((NC, _BLK, D), lambda i: (0, i, 0)),
            pl.BlockSpec((_BLK, D), lambda i: (i, 0)),
        ],
        out_specs=pl.BlockSpec((_BLK, D), lambda i: (i, 0)),
        out_shape=jax.ShapeDtypeStruct((N_PAD, D), jnp.float32),
    )(parts, total)


def kernel(playlist_weight, track_weight, edge_index, edge_weight):
    emb0 = jnp.zeros((N_PAD, D), jnp.float32)
    emb0 = emb0.at[:NUM_PL].set(playlist_weight)
    emb0 = emb0.at[NUM_PL:N].set(track_weight)
    src = edge_index[0].astype(jnp.int32)
    dst = edge_index[1].astype(jnp.int32)
    w = edge_weight.astype(jnp.float32)
    pad = E_PAD - E
    spread = (jnp.arange(pad, dtype=jnp.int32) * 8) % N_PAD
    src_p = jnp.concatenate([src, spread]).reshape(NCHT, CHUNK)
    dst_p = jnp.concatenate([dst, spread]).reshape(NCHT, CHUNK)
    w_p = jnp.concatenate(
        [w, jnp.zeros((pad,), jnp.float32)]).reshape(NCHT, CHUNK)

    emb = emb0
    total = emb0
    mean = None
    for layer in range(3):
        parts = _propagate(emb, src_p, dst_p, w_p)
        if layer < 2:
            emb, total = _combine(parts, total)
        else:
            mean = _final(parts, total)
    return mean[:NUM_PL], mean[NUM_PL:N]
